# Initial kernel scaffold; baseline (speedup 1.0000x reference)
import jax
import jax.numpy as jnp
from jax.experimental import pallas as pl


def kernel(idx, pe):
    raise NotImplementedError("write your pallas kernel here")


# speedup vs baseline: 3.9729x; 3.9729x over previous
import functools

import jax
import jax.numpy as jnp
from jax import lax
from jax.experimental import pallas as pl
from jax.experimental.pallas import tpu as pltpu
from jax.experimental.pallas import tpu_sc as plsc

MAX_ROWS = 100000
EMB_DIM = 64


def _gather_kernel(B, D, chunk):
    info = plsc.get_sparse_core_info()
    NC, NS = info.num_cores, info.num_subcores
    NW = NC * NS
    assert B % (NW * chunk) == 0
    b_per_w = B // NW
    n_chunks = b_per_w // chunk

    mesh = plsc.VectorSubcoreMesh(core_axis_name="c", subcore_axis_name="s")

    @functools.partial(
        pl.kernel,
        mesh=mesh,
        out_type=jax.ShapeDtypeStruct((B, D), jnp.float32),
        scratch_types=[
            pltpu.VMEM((chunk,), jnp.int32),
            pltpu.VMEM((chunk, D), jnp.float32),
            pltpu.SemaphoreType.DMA,
        ],
        compiler_params=pltpu.CompilerParams(use_tc_tiling_on_sc=False),
    )
    def k(idx_hbm, pe_hbm, out_hbm, idx_v, rows_v, sem):
        wid = lax.axis_index("s") * NC + lax.axis_index("c")
        base = wid * b_per_w

        def body(c, carry):
            off = base + c * chunk
            pltpu.sync_copy(idx_hbm.at[pl.ds(off, chunk)], idx_v)
            pltpu.async_copy(pe_hbm.at[idx_v], rows_v, sem).wait()
            pltpu.sync_copy(rows_v, out_hbm.at[pl.ds(off, chunk)])
            return carry

        lax.fori_loop(0, n_chunks, body, 0)

    return k


def kernel(idx, pe):
    B, T = idx.shape
    D = pe.shape[1]
    flat_idx = idx.reshape(B * T).astype(jnp.int32)
    out = _gather_kernel(B * T, D, 1024)(flat_idx, pe)
    return out.reshape(B, T, D)


# speedup vs baseline: 4.0562x; 1.0210x over previous
import functools

import jax
import jax.numpy as jnp
from jax import lax
from jax.experimental import pallas as pl
from jax.experimental.pallas import tpu as pltpu
from jax.experimental.pallas import tpu_sc as plsc


def _gather_kernel(B, D, chunk):
    info = plsc.get_sparse_core_info()
    NC, NS = info.num_cores, info.num_subcores
    NW = NC * NS
    assert B % (NW * 2 * chunk) == 0
    b_per_w = B // NW
    n2 = b_per_w // (2 * chunk)

    mesh = plsc.VectorSubcoreMesh(core_axis_name="c", subcore_axis_name="s")

    @functools.partial(
        pl.kernel,
        mesh=mesh,
        out_type=jax.ShapeDtypeStruct((B, D), jnp.float32),
        scratch_types=[
            pltpu.VMEM((b_per_w,), jnp.int32),
            pltpu.VMEM((2, chunk, D), jnp.float32),
            pltpu.SemaphoreType.DMA,
            pltpu.SemaphoreType.DMA,
            pltpu.SemaphoreType.DMA,
            pltpu.SemaphoreType.DMA,
        ],
        compiler_params=pltpu.CompilerParams(use_tc_tiling_on_sc=False),
    )
    def k(idx_hbm, pe_hbm, out_hbm, idx_v, rows_v, sg0, sg1, so0, so1):
        wid = lax.axis_index("s") * NC + lax.axis_index("c")
        base = wid * b_per_w
        pltpu.sync_copy(idx_hbm.at[pl.ds(base, b_per_w)], idx_v)

        def gather_desc(c, buf, sem):
            return pltpu.make_async_copy(
                pe_hbm.at[idx_v.at[pl.ds(c * chunk, chunk)]],
                rows_v.at[buf],
                sem,
            )

        def store_desc(c, buf, sem):
            return pltpu.make_async_copy(
                rows_v.at[buf],
                out_hbm.at[pl.ds(base + c * chunk, chunk)],
                sem,
            )

        gather_desc(0, 0, sg0).start()
        gather_desc(1, 1, sg1).start()
        gather_desc(0, 0, sg0).wait()
        store_desc(0, 0, so0).start()
        gather_desc(1, 1, sg1).wait()
        store_desc(1, 1, so1).start()

        def body(p, carry):
            c0 = 2 * p
            store_desc(c0 - 2, 0, so0).wait()
            gather_desc(c0, 0, sg0).start()
            store_desc(c0 - 1, 1, so1).wait()
            gather_desc(c0 + 1, 1, sg1).start()
            gather_desc(c0, 0, sg0).wait()
            store_desc(c0, 0, so0).start()
            gather_desc(c0 + 1, 1, sg1).wait()
            store_desc(c0 + 1, 1, so1).start()
            return carry

        lax.fori_loop(1, n2, body, 0)
        store_desc(2 * n2 - 2, 0, so0).wait()
        store_desc(2 * n2 - 1, 1, so1).wait()

    return k


def kernel(idx, pe):
    B, T = idx.shape
    D = pe.shape[1]
    flat_idx = idx.reshape(B * T).astype(jnp.int32)
    out = _gather_kernel(B * T, D, 640)(flat_idx, pe)
    return out.reshape(B, T, D)
